# baseline (speedup 1.0000x reference)
import functools

import jax
import jax.numpy as jnp
from jax import lax
from jax.experimental import pallas as pl
from jax.experimental.pallas import tpu as pltpu
from jax.experimental.pallas import tpu_sc as plsc

_B, _Q, _N, _C = 8, 8192, 4096, 128
_HID, _OUT = 64, 3
_W = 128

_NW = 32
_NS = 4
_HB = _B // _NS
_ROWS = 2 * _HB * _Q
_RPW = _ROWS // _NW
_CH = 128
_NCH = _RPW // _CH
_GCH = 2
_NGRP = _NCH // _GCH
_GROWS = _GCH * _CH

_QB = 4096


def _fc_val(i):
    fi = i.astype(jnp.float32)
    return (2.0 * fi - float(_N - 1)) * (1.0 / _N)


def _prep_body(coord_ref, lat_ref, w0f_ref, b0_ref, t_ref, idx_ref, rel_ref):
    lat = lat_ref[0]
    p = jnp.dot(lat, w0f_ref[0:_C, :], preferred_element_type=jnp.float32)
    c = jnp.dot(lat, w0f_ref[_C:2 * _C, :], preferred_element_type=jnp.float32)
    n = jnp.dot(lat, w0f_ref[2 * _C:3 * _C, :], preferred_element_type=jnp.float32)
    prev = jnp.concatenate([p[0:1], p[:-1]], axis=0)
    nxt = jnp.concatenate([n[1:], n[-1:]], axis=0)
    t_ref[0] = prev + c + nxt + b0_ref[...]

    cq = coord_ref[0]
    b = pl.program_id(0)
    rx = 2.0 / _N / 2.0
    for k, vx in enumerate((-2, 0)):
        cs = cq + vx * rx + 1e-06
        cs = jnp.clip(cs, -1.0 + 1e-06, 1.0 - 1e-06)
        u = cs * float(_N)
        e = jnp.floor((u + float(_N - 1)) * 0.5).astype(jnp.int32)
        e = jnp.clip(e, 0, _N)
        for _ in range(2):
            up = (e < _N) & (_fc_val(jnp.minimum(e, _N - 1)) < cs)
            e = e + up.astype(jnp.int32)
        for _ in range(2):
            dn = (e > 0) & (_fc_val(jnp.maximum(e, 1) - 1) >= cs)
            e = e - dn.astype(jnp.int32)
        idx = jnp.clip(e, 0, _N - 1)
        rel = (cq - _fc_val(idx)) * float(_N)
        idx_ref[0, k] = idx + b * _N
        rel_ref[0, k] = rel


def _sc_gather_body(table_hbm, idx_hbm, rel_hbm, out_hbm,
                    idx_v, rel_v, buf_a, buf_b, sem_a, sem_b):
    wid = lax.axis_index("s") * 2 + lax.axis_index("c")
    base = wid * _RPW
    pltpu.sync_copy(idx_hbm.at[wid], idx_v)
    pltpu.sync_copy(rel_hbm.at[wid], rel_v)

    bufs = (buf_a, buf_b)
    sems = (sem_a, sem_b)
    col64 = jnp.full((16,), _HID, jnp.int32)

    def fire(g, buf, sem):
        return [
            pltpu.async_copy(
                table_hbm.at[idx_v.at[g * _GCH + k]],
                buf.at[pl.ds(k * _CH, _CH)],
                sem,
            )
            for k in range(_GCH)
        ]

    def stash_rel(g, buf):
        for k in range(_GCH):
            j = g * _GCH + k
            for s in range(_CH // 16):
                rv = rel_v[j, pl.ds(s * 16, 16)]
                rows = lax.iota(jnp.int32, 16) + (k * _CH + s * 16)
                plsc.store_scatter(buf, [rows, col64], rv)

    pending = {0: fire(0, bufs[0], sems[0])}
    for g in range(_NGRP):
        if g + 1 < _NGRP:
            pending[(g + 1) % 2] = fire(g + 1, bufs[(g + 1) % 2], sems[(g + 1) % 2])
        for cp in pending[g % 2]:
            cp.wait()
        stash_rel(g, bufs[g % 2])
        pltpu.sync_copy(bufs[g % 2], out_hbm.at[pl.ds(base + g * _GROWS, _GROWS)])


@functools.lru_cache(maxsize=None)
def _sc_gather_call():
    return pl.kernel(
        _sc_gather_body,
        out_type=jax.ShapeDtypeStruct((_ROWS, _W), jnp.float32),
        mesh=plsc.VectorSubcoreMesh(core_axis_name="c", subcore_axis_name="s"),
        scratch_types=[
            pltpu.VMEM((_NCH, _CH), jnp.int32),
            pltpu.VMEM((_NCH, _CH), jnp.float32),
            pltpu.VMEM((_GROWS, _W), jnp.float32),
            pltpu.VMEM((_GROWS, _W), jnp.float32),
            pltpu.SemaphoreType.DMA,
            pltpu.SemaphoreType.DMA,
        ],
        compiler_params=pltpu.CompilerParams(needs_layout_passes=False),
    )


def _mlp_body(g_ref, w0r_ref, w1_ref, b1_ref, out_ref):
    g0 = g_ref[0, 0]
    g1 = g_ref[0, 1]
    r0 = g0[:, _HID:_HID + 1]
    r1 = g1[:, _HID:_HID + 1]
    w0r = w0r_ref[...]
    h0 = jnp.maximum(g0 + r0 * w0r, 0.0)
    h1 = jnp.maximum(g1 + r1 * w0r, 0.0)
    a0 = jnp.abs(r0) + 1e-09
    a1 = jnp.abs(r1) + 1e-09
    tot = a0 + a1
    hb = h0 * (a1 / tot) + h1 * (a0 / tot)
    res = lax.dot_general(w1_ref[...], hb, (((0,), (1,)), ((), ())),
                          preferred_element_type=jnp.float32)
    out_ref[:, 0, 0] = res + b1_ref[...]


def kernel(coord, latent, W0, b0, W1, b1):
    coord3 = coord.reshape(_B, 1, _Q)
    pad = ((0, 0), (0, _W - _HID))
    w0f = jnp.pad(W0[1:, :], pad)
    w0r = jnp.pad(W0[0:1, :], pad)
    b0p = jnp.pad(b0.reshape(1, _HID), pad)
    w1p = jnp.pad(W1, ((0, _W - _HID), (0, 0)))
    b1c = b1.reshape(_OUT, 1)

    t, idxg, rel = pl.pallas_call(
        _prep_body,
        grid=(_B,),
        in_specs=[
            pl.BlockSpec((1, 1, _Q), lambda b: (b, 0, 0)),
            pl.BlockSpec((1, _N, _C), lambda b: (b, 0, 0)),
            pl.BlockSpec((3 * _C, _W), lambda b: (0, 0)),
            pl.BlockSpec((1, _W), lambda b: (0, 0)),
        ],
        out_specs=[
            pl.BlockSpec((1, _N, _W), lambda b: (b, 0, 0)),
            pl.BlockSpec((1, 2, 1, _Q), lambda b: (b, 0, 0, 0)),
            pl.BlockSpec((1, 2, 1, _Q), lambda b: (b, 0, 0, 0)),
        ],
        out_shape=[
            jax.ShapeDtypeStruct((_B, _N, _W), jnp.float32),
            jax.ShapeDtypeStruct((_B, 2, 1, _Q), jnp.int32),
            jax.ShapeDtypeStruct((_B, 2, 1, _Q), jnp.float32),
        ],
    )(coord3, latent, w0f, b0p)

    tt = t.reshape(_B * _N, _W)
    idxh = idxg.reshape(_NS, _NW, _NCH, _CH)
    relh = rel.reshape(_NS, _NW, _NCH, _CH)

    def blend(gh):
        return pl.pallas_call(
            _mlp_body,
            grid=(_HB, _Q // _QB),
            in_specs=[
                pl.BlockSpec((1, 2, _QB, _W), lambda b, q: (b, 0, q, 0)),
                pl.BlockSpec((1, _W), lambda b, q: (0, 0)),
                pl.BlockSpec((_W, _OUT), lambda b, q: (0, 0)),
                pl.BlockSpec((_OUT, 1), lambda b, q: (0, 0)),
            ],
            out_specs=pl.BlockSpec((_OUT, 1, 1, _QB), lambda b, q: (0, b, 0, q)),
            out_shape=jax.ShapeDtypeStruct((_OUT, _HB, 1, _Q), jnp.float32),
        )(gh.reshape(_HB, 2, _Q, _W), w0r, w1p, b1c)

    sc = _sc_gather_call()
    gs = [sc(tt, idxh[s], relh[s]) for s in range(_NS)]
    outs = [blend(g) for g in gs]
    out = jnp.concatenate(outs, axis=1).reshape(_OUT, _B, _Q)
    return jnp.transpose(out, (1, 2, 0))


# speedup vs baseline: 1.0417x; 1.0417x over previous
import functools

import jax
import jax.numpy as jnp
from jax import lax
from jax.experimental import pallas as pl
from jax.experimental.pallas import tpu as pltpu
from jax.experimental.pallas import tpu_sc as plsc

_B, _Q, _N, _C = 8, 8192, 4096, 128
_HID, _OUT = 64, 3
_W = 128

_NW = 32
_HB = _B // 2
_ROWS = 2 * _HB * _Q
_RPW = _ROWS // _NW
_CH = 128
_NCH = _RPW // _CH
_GCH = 2
_NGRP = _NCH // _GCH
_GROWS = _GCH * _CH

_QB = 4096


def _fc_val(i):
    fi = i.astype(jnp.float32)
    return (2.0 * fi - float(_N - 1)) * (1.0 / _N)


def _prep_body(coord_ref, lat_ref, w0f_ref, b0_ref, t_ref, idx_ref, rel_ref):
    lat = lat_ref[0]
    p = jnp.dot(lat, w0f_ref[0:_C, :], preferred_element_type=jnp.float32)
    c = jnp.dot(lat, w0f_ref[_C:2 * _C, :], preferred_element_type=jnp.float32)
    n = jnp.dot(lat, w0f_ref[2 * _C:3 * _C, :], preferred_element_type=jnp.float32)
    prev = jnp.concatenate([p[0:1], p[:-1]], axis=0)
    nxt = jnp.concatenate([n[1:], n[-1:]], axis=0)
    t_ref[0] = prev + c + nxt + b0_ref[...]

    cq = coord_ref[0]
    b = pl.program_id(0)
    rx = 2.0 / _N / 2.0
    for k, vx in enumerate((-2, 0)):
        cs = cq + vx * rx + 1e-06
        cs = jnp.clip(cs, -1.0 + 1e-06, 1.0 - 1e-06)
        u = cs * float(_N)
        e = jnp.floor((u + float(_N - 1)) * 0.5).astype(jnp.int32)
        e = jnp.clip(e, 0, _N)
        for _ in range(2):
            up = (e < _N) & (_fc_val(jnp.minimum(e, _N - 1)) < cs)
            e = e + up.astype(jnp.int32)
        for _ in range(2):
            dn = (e > 0) & (_fc_val(jnp.maximum(e, 1) - 1) >= cs)
            e = e - dn.astype(jnp.int32)
        idx = jnp.clip(e, 0, _N - 1)
        rel = (cq - _fc_val(idx)) * float(_N)
        idx_ref[0, k] = idx + b * _N
        rel_ref[0, k] = rel


def _sc_gather_body(table_hbm, idx_hbm, rel_hbm, out_hbm,
                    idx_v, rel_v, buf_a, buf_b, sem_a, sem_b):
    wid = lax.axis_index("s") * 2 + lax.axis_index("c")
    base = wid * _RPW
    pltpu.sync_copy(idx_hbm.at[wid], idx_v)
    pltpu.sync_copy(rel_hbm.at[wid], rel_v)

    bufs = (buf_a, buf_b)
    sems = (sem_a, sem_b)
    col64 = jnp.full((16,), _HID, jnp.int32)

    def fire(g, buf, sem):
        return [
            pltpu.async_copy(
                table_hbm.at[idx_v.at[g * _GCH + k]],
                buf.at[pl.ds(k * _CH, _CH)],
                sem,
            )
            for k in range(_GCH)
        ]

    def stash_rel(g, buf):
        for k in range(_GCH):
            j = g * _GCH + k
            for s in range(_CH // 16):
                rv = rel_v[j, pl.ds(s * 16, 16)]
                rows = lax.iota(jnp.int32, 16) + (k * _CH + s * 16)
                plsc.store_scatter(buf, [rows, col64], rv)

    pending = {0: fire(0, bufs[0], sems[0])}
    for g in range(_NGRP):
        if g + 1 < _NGRP:
            pending[(g + 1) % 2] = fire(g + 1, bufs[(g + 1) % 2], sems[(g + 1) % 2])
        for cp in pending[g % 2]:
            cp.wait()
        stash_rel(g, bufs[g % 2])
        pltpu.sync_copy(bufs[g % 2], out_hbm.at[pl.ds(base + g * _GROWS, _GROWS)])


@functools.lru_cache(maxsize=None)
def _sc_gather_call():
    return pl.kernel(
        _sc_gather_body,
        out_type=jax.ShapeDtypeStruct((_ROWS, _W), jnp.float32),
        mesh=plsc.VectorSubcoreMesh(core_axis_name="c", subcore_axis_name="s"),
        scratch_types=[
            pltpu.VMEM((_NCH, _CH), jnp.int32),
            pltpu.VMEM((_NCH, _CH), jnp.float32),
            pltpu.VMEM((_GROWS, _W), jnp.float32),
            pltpu.VMEM((_GROWS, _W), jnp.float32),
            pltpu.SemaphoreType.DMA,
            pltpu.SemaphoreType.DMA,
        ],
        compiler_params=pltpu.CompilerParams(needs_layout_passes=False),
    )


def _mlp_body(g_ref, w0r_ref, w1_ref, b1_ref, out_ref):
    g0 = g_ref[0, 0]
    g1 = g_ref[0, 1]
    r0 = g0[:, _HID:_HID + 1]
    r1 = g1[:, _HID:_HID + 1]
    w0r = w0r_ref[...]
    h0 = jnp.maximum(g0 + r0 * w0r, 0.0)
    h1 = jnp.maximum(g1 + r1 * w0r, 0.0)
    a0 = jnp.abs(r0) + 1e-09
    a1 = jnp.abs(r1) + 1e-09
    tot = a0 + a1
    hb = h0 * (a1 / tot) + h1 * (a0 / tot)
    res = lax.dot_general(w1_ref[...], hb, (((0,), (1,)), ((), ())),
                          preferred_element_type=jnp.float32)
    out_ref[:, 0, 0] = res + b1_ref[...]


def kernel(coord, latent, W0, b0, W1, b1):
    coord3 = coord.reshape(_B, 1, _Q)
    pad = ((0, 0), (0, _W - _HID))
    w0f = jnp.pad(W0[1:, :], pad)
    w0r = jnp.pad(W0[0:1, :], pad)
    b0p = jnp.pad(b0.reshape(1, _HID), pad)
    w1p = jnp.pad(W1, ((0, _W - _HID), (0, 0)))
    b1c = b1.reshape(_OUT, 1)

    t, idxg, rel = pl.pallas_call(
        _prep_body,
        grid=(_B,),
        in_specs=[
            pl.BlockSpec((1, 1, _Q), lambda b: (b, 0, 0)),
            pl.BlockSpec((1, _N, _C), lambda b: (b, 0, 0)),
            pl.BlockSpec((3 * _C, _W), lambda b: (0, 0)),
            pl.BlockSpec((1, _W), lambda b: (0, 0)),
        ],
        out_specs=[
            pl.BlockSpec((1, _N, _W), lambda b: (b, 0, 0)),
            pl.BlockSpec((1, 2, 1, _Q), lambda b: (b, 0, 0, 0)),
            pl.BlockSpec((1, 2, 1, _Q), lambda b: (b, 0, 0, 0)),
        ],
        out_shape=[
            jax.ShapeDtypeStruct((_B, _N, _W), jnp.float32),
            jax.ShapeDtypeStruct((_B, 2, 1, _Q), jnp.int32),
            jax.ShapeDtypeStruct((_B, 2, 1, _Q), jnp.float32),
        ],
    )(coord3, latent, w0f, b0p)

    tt = t.reshape(_B * _N, _W)
    idxh = idxg.reshape(2, _NW, _NCH, _CH)
    relh = rel.reshape(2, _NW, _NCH, _CH)

    def blend(gh):
        return pl.pallas_call(
            _mlp_body,
            grid=(_HB, _Q // _QB),
            in_specs=[
                pl.BlockSpec((1, 2, _QB, _W), lambda b, q: (b, 0, q, 0)),
                pl.BlockSpec((1, _W), lambda b, q: (0, 0)),
                pl.BlockSpec((_W, _OUT), lambda b, q: (0, 0)),
                pl.BlockSpec((_OUT, 1), lambda b, q: (0, 0)),
            ],
            out_specs=pl.BlockSpec((_OUT, 1, 1, _QB), lambda b, q: (0, b, 0, q)),
            out_shape=jax.ShapeDtypeStruct((_OUT, _HB, 1, _Q), jnp.float32),
        )(gh.reshape(_HB, 2, _Q, _W), w0r, w1p, b1c)

    sc = _sc_gather_call()
    g0 = sc(tt, idxh[0], relh[0])
    g1 = sc(tt, idxh[1], relh[1])
    out0 = blend(g0)
    out1 = blend(g1)
    out = jnp.concatenate([out0, out1], axis=1).reshape(_OUT, _B, _Q)
    return jnp.transpose(out, (1, 2, 0))
